# baseline (speedup 1.0000x reference)
import jax
import jax.numpy as jnp
from jax import lax
from jax.experimental import pallas as pl
from jax.experimental.pallas import tpu as pltpu
from jax.experimental.pallas import tpu_sc as plsc

N_NODES = 10000
N_EDGES = 320000
D_FEAT = 128
L = 16
NW = 32
NSUB = 16
E_TILE = N_EDGES // NW
CHUNK = 80
NCHUNK = E_TILE // CHUNK
NGROUP = CHUNK // L
IB = 4
NBLK = 32
ROW_STRIDE = 624
ROWS_PER_SUB = 640


def _rsqrt_nr(x):
    i = plsc.bitcast(x, jnp.int32)
    i = jnp.int32(0x5F3759DF) - lax.shift_right_logical(i, 1)
    y = plsc.bitcast(i, jnp.float32)
    for _ in range(3):
        y = y * (jnp.float32(1.5) - jnp.float32(0.5) * x * y * y)
    return y


def _edge_cosine_body(h_hbm, src_hbm, dst_hbm, out_hbm,
                      h_sh,
                      idx_s_v, idx_d_v, rows_s_v, rows_d_v, out_v,
                      sem_i0, sem_i1, sem_r0, sem_r1, sem_o0, sem_o1):
    cid = lax.axis_index("c")
    sid = lax.axis_index("s")
    wid = sid * 2 + cid
    sem_i = (sem_i0, sem_i1)
    sem_r = (sem_r0, sem_r1)
    sem_o = (sem_o0, sem_o1)
    out0 = pl.multiple_of(wid * E_TILE, 16)

    def fetch_idx(blk, slot):
        pltpu.async_copy(src_hbm.at[wid, blk], idx_s_v.at[slot], sem_i[slot])
        pltpu.async_copy(dst_hbm.at[wid, blk], idx_d_v.at[slot], sem_i[slot])

    def wait_idx(slot):
        pltpu.make_async_copy(
            src_hbm.at[wid, 0], idx_s_v.at[slot], sem_i[slot]).wait()
        pltpu.make_async_copy(
            dst_hbm.at[wid, 0], idx_d_v.at[slot], sem_i[slot]).wait()

    def start_rows(islot, k, rslot):
        pltpu.async_copy(
            h_sh.at[idx_s_v.at[islot, k]], rows_s_v.at[rslot], sem_r[rslot])
        pltpu.async_copy(
            h_sh.at[idx_d_v.at[islot, k]], rows_d_v.at[rslot], sem_r[rslot])

    def wait_rows(rslot):
        pltpu.make_async_copy(
            h_sh.at[idx_s_v.at[0, 0]], rows_s_v.at[rslot], sem_r[rslot]).wait()
        pltpu.make_async_copy(
            h_sh.at[idx_d_v.at[0, 0]], rows_d_v.at[rslot], sem_r[rslot]).wait()

    def compute(c, rslot):
        rs = rows_s_v.at[rslot]
        rd = rows_d_v.at[rslot]
        ov = out_v.at[rslot]
        for g in range(NGROUP):
            e16 = lax.iota(jnp.int32, L) + (g * L)

            def f_body(f, acc):
                del f
                dot, qs, qd, fv = acc
                s = plsc.load_gather(rs, [e16, fv])
                d = plsc.load_gather(rd, [e16, fv])
                return (dot + s * d, qs + s * s, qd + d * d, fv + 1)

            zeros = jnp.zeros((L,), jnp.float32)
            dot, qs, qd, _ = lax.fori_loop(
                0, D_FEAT, f_body,
                (zeros, zeros, zeros, jnp.zeros((L,), jnp.int32)),
                unroll=8)
            denom2 = jnp.maximum(qs * qd, jnp.float32(1e-16))
            res = jnp.maximum(dot * _rsqrt_nr(denom2), jnp.float32(0.0))
            ov[pl.ds(g * L, L)] = res
        pltpu.async_copy(
            out_v.at[rslot],
            out_hbm.at[pl.ds(out0 + c * CHUNK, CHUNK)], sem_o[rslot])

    def wait_out(rslot):
        pltpu.make_async_copy(
            out_v.at[rslot], out_hbm.at[pl.ds(out0, CHUNK)],
            sem_o[rslot]).wait()

    pltpu.sync_copy(src_hbm.at[wid, 0], idx_s_v.at[0])
    pltpu.sync_copy(dst_hbm.at[wid, 0], idx_d_v.at[0])
    fetch_idx(1, 1)

    row0 = pl.multiple_of(sid * ROW_STRIDE, 16)
    pltpu.sync_copy(h_hbm.at[pl.ds(row0, ROWS_PER_SUB)],
                    h_sh.at[pl.ds(row0, ROWS_PER_SUB)])
    plsc.subcore_barrier()

    start_rows(0, 0, 0)

    def pair_body(J, carry):
        for m in range(2 * IB):
            c = 2 * IB * J + m
            islot, k = ((m + 1) // IB) % 2, (m + 1) % IB

            if m == IB - 1:
                @pl.when(c + 1 < NCHUNK)
                def _():
                    wait_idx(1)
            if m == 2 * IB - 1:
                @pl.when(c + 1 < NCHUNK)
                def _():
                    wait_idx(0)

            @pl.when(c + 1 < NCHUNK)
            def _():
                start_rows(islot, k, (m + 1) % 2)

            @pl.when(c < NCHUNK)
            def _():
                wait_rows(m % 2)

            @pl.when(jnp.logical_and(2 <= c, c < NCHUNK))
            def _():
                wait_out(m % 2)

            @pl.when(c < NCHUNK)
            def _():
                compute(c, m % 2)

            if m == IB - 1:
                @pl.when(2 * J + 2 < NBLK)
                def _():
                    fetch_idx(2 * J + 2, 0)
            if m == 2 * IB - 1:
                @pl.when(2 * J + 3 < NBLK)
                def _():
                    fetch_idx(2 * J + 3, 1)
        return carry

    lax.fori_loop(0, NBLK // 2, pair_body, jnp.int32(0))
    wait_out(1)
    wait_out(0)


def kernel(h, src, dst):
    pad = NBLK * IB * CHUNK - E_TILE
    src4 = jnp.pad(src.reshape(NW, E_TILE), ((0, 0), (0, pad))
                   ).reshape(NW, NBLK, IB, CHUNK)
    dst4 = jnp.pad(dst.reshape(NW, E_TILE), ((0, 0), (0, pad))
                   ).reshape(NW, NBLK, IB, CHUNK)
    run = pl.kernel(
        _edge_cosine_body,
        mesh=plsc.VectorSubcoreMesh(core_axis_name="c", subcore_axis_name="s"),
        out_type=jax.ShapeDtypeStruct((N_EDGES,), jnp.float32),
        scratch_types=[
            pltpu.VMEM_SHARED((N_NODES, D_FEAT), jnp.float32),
            pltpu.VMEM((2, IB, CHUNK), jnp.int32),
            pltpu.VMEM((2, IB, CHUNK), jnp.int32),
            pltpu.VMEM((2, CHUNK, D_FEAT), jnp.float32),
            pltpu.VMEM((2, CHUNK, D_FEAT), jnp.float32),
            pltpu.VMEM((2, CHUNK), jnp.float32),
            pltpu.SemaphoreType.DMA,
            pltpu.SemaphoreType.DMA,
            pltpu.SemaphoreType.DMA,
            pltpu.SemaphoreType.DMA,
            pltpu.SemaphoreType.DMA,
            pltpu.SemaphoreType.DMA,
        ],
        compiler_params=pltpu.CompilerParams(needs_layout_passes=False),
    )
    return run(h, src4, dst4)


# speedup vs baseline: 1.0453x; 1.0453x over previous
import jax
import jax.numpy as jnp
from jax import lax
from jax.experimental import pallas as pl
from jax.experimental.pallas import tpu as pltpu
from jax.experimental.pallas import tpu_sc as plsc

N_NODES = 10000
N_EDGES = 320000
D_FEAT = 128
L = 16
NW = 32
NSUB = 16
E_TILE = N_EDGES // NW
CHUNK = 80
NCHUNK = E_TILE // CHUNK
NGROUP = CHUNK // L
IB = 4
NBLK = 32
ROW_STRIDE = 624
ROWS_PER_SUB = 640


def _rsqrt_nr(x):
    i = plsc.bitcast(x, jnp.int32)
    i = jnp.int32(0x5F3759DF) - lax.shift_right_logical(i, 1)
    y = plsc.bitcast(i, jnp.float32)
    for _ in range(3):
        y = y * (jnp.float32(1.5) - jnp.float32(0.5) * x * y * y)
    return y


def _edge_cosine_body(h_hbm, src_hbm, dst_hbm, out_hbm,
                      h_sh,
                      idx_s_v, idx_d_v, rows_s_v, rows_d_v, out_v,
                      sem_i0, sem_i1, sem_r0, sem_r1, sem_o0, sem_o1):
    cid = lax.axis_index("c")
    sid = lax.axis_index("s")
    wid = sid * 2 + cid
    sem_i = (sem_i0, sem_i1)
    sem_r = (sem_r0, sem_r1)
    sem_o = (sem_o0, sem_o1)
    out0 = pl.multiple_of(wid * E_TILE, 16)

    def fetch_idx(blk, slot):
        pltpu.async_copy(src_hbm.at[wid, blk], idx_s_v.at[slot], sem_i[slot])
        pltpu.async_copy(dst_hbm.at[wid, blk], idx_d_v.at[slot], sem_i[slot])

    def wait_idx(slot):
        pltpu.make_async_copy(
            src_hbm.at[wid, 0], idx_s_v.at[slot], sem_i[slot]).wait()
        pltpu.make_async_copy(
            dst_hbm.at[wid, 0], idx_d_v.at[slot], sem_i[slot]).wait()

    def start_rows(islot, k, rslot):
        pltpu.async_copy(
            h_sh.at[idx_s_v.at[islot, k]], rows_s_v.at[rslot], sem_r[rslot])
        pltpu.async_copy(
            h_sh.at[idx_d_v.at[islot, k]], rows_d_v.at[rslot], sem_r[rslot])

    def wait_rows(rslot):
        pltpu.make_async_copy(
            h_sh.at[idx_s_v.at[0, 0]], rows_s_v.at[rslot], sem_r[rslot]).wait()
        pltpu.make_async_copy(
            h_sh.at[idx_d_v.at[0, 0]], rows_d_v.at[rslot], sem_r[rslot]).wait()

    def compute(c, rslot):
        rs = rows_s_v.at[rslot]
        rd = rows_d_v.at[rslot]
        ov = out_v.at[rslot]
        ebase = lax.iota(jnp.int32, L) * D_FEAT
        z16 = jnp.zeros((L,), jnp.int32)
        for g in range(NGROUP):
            addr0 = ebase + (g * L * D_FEAT)

            def f2_body(f, acc):
                del f
                dot0, dot1, qs0, qs1, qd0, qd1, a = acc
                s0 = plsc.load_gather(rs, [z16, a])
                d0 = plsc.load_gather(rd, [z16, a])
                a1 = a + 1
                s1 = plsc.load_gather(rs, [z16, a1])
                d1 = plsc.load_gather(rd, [z16, a1])
                return (dot0 + s0 * d0, dot1 + s1 * d1,
                        qs0 + s0 * s0, qs1 + s1 * s1,
                        qd0 + d0 * d0, qd1 + d1 * d1, a + 2)

            z = jnp.zeros((L,), jnp.float32)
            dot0, dot1, qs0, qs1, qd0, qd1, _ = lax.fori_loop(
                0, D_FEAT // 2, f2_body, (z, z, z, z, z, z, addr0),
                unroll=4)
            dot = dot0 + dot1
            denom2 = jnp.maximum((qs0 + qs1) * (qd0 + qd1), jnp.float32(1e-16))
            res = jnp.maximum(dot * _rsqrt_nr(denom2), jnp.float32(0.0))
            ov[pl.ds(g * L, L)] = res
        pltpu.async_copy(
            out_v.at[rslot],
            out_hbm.at[pl.ds(out0 + c * CHUNK, CHUNK)], sem_o[rslot])

    def wait_out(rslot):
        pltpu.make_async_copy(
            out_v.at[rslot], out_hbm.at[pl.ds(out0, CHUNK)],
            sem_o[rslot]).wait()

    pltpu.sync_copy(src_hbm.at[wid, 0], idx_s_v.at[0])
    pltpu.sync_copy(dst_hbm.at[wid, 0], idx_d_v.at[0])
    fetch_idx(1, 1)

    row0 = pl.multiple_of(sid * ROW_STRIDE, 16)
    pltpu.sync_copy(h_hbm.at[pl.ds(row0, ROWS_PER_SUB)],
                    h_sh.at[pl.ds(row0, ROWS_PER_SUB)])
    plsc.subcore_barrier()

    start_rows(0, 0, 0)

    def pair_body(J, carry):
        for m in range(2 * IB):
            c = 2 * IB * J + m
            islot, k = ((m + 1) // IB) % 2, (m + 1) % IB

            if m == IB - 1:
                @pl.when(c + 1 < NCHUNK)
                def _():
                    wait_idx(1)
            if m == 2 * IB - 1:
                @pl.when(c + 1 < NCHUNK)
                def _():
                    wait_idx(0)

            @pl.when(c + 1 < NCHUNK)
            def _():
                start_rows(islot, k, (m + 1) % 2)

            @pl.when(c < NCHUNK)
            def _():
                wait_rows(m % 2)

            @pl.when(jnp.logical_and(2 <= c, c < NCHUNK))
            def _():
                wait_out(m % 2)

            @pl.when(c < NCHUNK)
            def _():
                compute(c, m % 2)

            if m == IB - 1:
                @pl.when(2 * J + 2 < NBLK)
                def _():
                    fetch_idx(2 * J + 2, 0)
            if m == 2 * IB - 1:
                @pl.when(2 * J + 3 < NBLK)
                def _():
                    fetch_idx(2 * J + 3, 1)
        return carry

    lax.fori_loop(0, NBLK // 2, pair_body, jnp.int32(0))
    wait_out(1)
    wait_out(0)


def kernel(h, src, dst):
    pad = NBLK * IB * CHUNK - E_TILE
    src4 = jnp.pad(src.reshape(NW, E_TILE), ((0, 0), (0, pad))
                   ).reshape(NW, NBLK, IB, CHUNK)
    dst4 = jnp.pad(dst.reshape(NW, E_TILE), ((0, 0), (0, pad))
                   ).reshape(NW, NBLK, IB, CHUNK)
    run = pl.kernel(
        _edge_cosine_body,
        mesh=plsc.VectorSubcoreMesh(core_axis_name="c", subcore_axis_name="s"),
        out_type=jax.ShapeDtypeStruct((N_EDGES,), jnp.float32),
        scratch_types=[
            pltpu.VMEM_SHARED((N_NODES, D_FEAT), jnp.float32),
            pltpu.VMEM((2, IB, CHUNK), jnp.int32),
            pltpu.VMEM((2, IB, CHUNK), jnp.int32),
            pltpu.VMEM((2, CHUNK, D_FEAT), jnp.float32),
            pltpu.VMEM((2, CHUNK, D_FEAT), jnp.float32),
            pltpu.VMEM((2, CHUNK), jnp.float32),
            pltpu.SemaphoreType.DMA,
            pltpu.SemaphoreType.DMA,
            pltpu.SemaphoreType.DMA,
            pltpu.SemaphoreType.DMA,
            pltpu.SemaphoreType.DMA,
            pltpu.SemaphoreType.DMA,
        ],
        compiler_params=pltpu.CompilerParams(needs_layout_passes=False),
    )
    return run(h, src4, dst4)


# speedup vs baseline: 1.0454x; 1.0001x over previous
import jax
import jax.numpy as jnp
from jax import lax
from jax.experimental import pallas as pl
from jax.experimental.pallas import tpu as pltpu
from jax.experimental.pallas import tpu_sc as plsc

N_NODES = 10000
N_EDGES = 320000
D_FEAT = 128
L = 16
NW = 32
NSUB = 16
E_TILE = N_EDGES // NW
CHUNK = 80
NCHUNK = E_TILE // CHUNK
NGROUP = CHUNK // L
IB = 4
NBLK = 32
ROW_STRIDE = 624
ROWS_PER_SUB = 640


def _rsqrt_nr(x):
    i = plsc.bitcast(x, jnp.int32)
    i = jnp.int32(0x5F3759DF) - lax.shift_right_logical(i, 1)
    y = plsc.bitcast(i, jnp.float32)
    for _ in range(3):
        y = y * (jnp.float32(1.5) - jnp.float32(0.5) * x * y * y)
    return y


def _edge_cosine_body(h_hbm, src_hbm, dst_hbm, out_hbm,
                      h_sh,
                      idx_s_v, idx_d_v, rows_s_v, rows_d_v, out_v,
                      sem_i0, sem_i1, sem_r0, sem_r1, sem_o0, sem_o1):
    cid = lax.axis_index("c")
    sid = lax.axis_index("s")
    wid = sid * 2 + cid
    sem_i = (sem_i0, sem_i1)
    sem_r = (sem_r0, sem_r1)
    sem_o = (sem_o0, sem_o1)
    out0 = pl.multiple_of(wid * E_TILE, 16)

    def fetch_idx(blk, slot):
        pltpu.async_copy(src_hbm.at[wid, blk], idx_s_v.at[slot], sem_i[slot])
        pltpu.async_copy(dst_hbm.at[wid, blk], idx_d_v.at[slot], sem_i[slot])

    def wait_idx(slot):
        pltpu.make_async_copy(
            src_hbm.at[wid, 0], idx_s_v.at[slot], sem_i[slot]).wait()
        pltpu.make_async_copy(
            dst_hbm.at[wid, 0], idx_d_v.at[slot], sem_i[slot]).wait()

    def start_rows(islot, k, rslot):
        hsrc = h_sh if rslot == 0 else h_hbm
        pltpu.async_copy(
            hsrc.at[idx_s_v.at[islot, k]], rows_s_v.at[rslot], sem_r[rslot])
        pltpu.async_copy(
            hsrc.at[idx_d_v.at[islot, k]], rows_d_v.at[rslot], sem_r[rslot])

    def wait_rows(rslot):
        hsrc = h_sh if rslot == 0 else h_hbm
        pltpu.make_async_copy(
            hsrc.at[idx_s_v.at[0, 0]], rows_s_v.at[rslot], sem_r[rslot]).wait()
        pltpu.make_async_copy(
            hsrc.at[idx_d_v.at[0, 0]], rows_d_v.at[rslot], sem_r[rslot]).wait()

    def compute(c, rslot):
        rs = rows_s_v.at[rslot]
        rd = rows_d_v.at[rslot]
        ov = out_v.at[rslot]
        ebase = lax.iota(jnp.int32, L) * D_FEAT
        z16 = jnp.zeros((L,), jnp.int32)
        for g in range(NGROUP):
            addr0 = ebase + (g * L * D_FEAT)

            def f2_body(f, acc):
                del f
                dot0, dot1, qs0, qs1, qd0, qd1, a = acc
                s0 = plsc.load_gather(rs, [z16, a])
                d0 = plsc.load_gather(rd, [z16, a])
                a1 = a + 1
                s1 = plsc.load_gather(rs, [z16, a1])
                d1 = plsc.load_gather(rd, [z16, a1])
                return (dot0 + s0 * d0, dot1 + s1 * d1,
                        qs0 + s0 * s0, qs1 + s1 * s1,
                        qd0 + d0 * d0, qd1 + d1 * d1, a + 2)

            z = jnp.zeros((L,), jnp.float32)
            dot0, dot1, qs0, qs1, qd0, qd1, _ = lax.fori_loop(
                0, D_FEAT // 2, f2_body, (z, z, z, z, z, z, addr0),
                unroll=4)
            dot = dot0 + dot1
            denom2 = jnp.maximum((qs0 + qs1) * (qd0 + qd1), jnp.float32(1e-16))
            res = jnp.maximum(dot * _rsqrt_nr(denom2), jnp.float32(0.0))
            ov[pl.ds(g * L, L)] = res
        pltpu.async_copy(
            out_v.at[rslot],
            out_hbm.at[pl.ds(out0 + c * CHUNK, CHUNK)], sem_o[rslot])

    def wait_out(rslot):
        pltpu.make_async_copy(
            out_v.at[rslot], out_hbm.at[pl.ds(out0, CHUNK)],
            sem_o[rslot]).wait()

    pltpu.sync_copy(src_hbm.at[wid, 0], idx_s_v.at[0])
    pltpu.sync_copy(dst_hbm.at[wid, 0], idx_d_v.at[0])
    fetch_idx(1, 1)

    row0 = pl.multiple_of(sid * ROW_STRIDE, 16)
    pltpu.sync_copy(h_hbm.at[pl.ds(row0, ROWS_PER_SUB)],
                    h_sh.at[pl.ds(row0, ROWS_PER_SUB)])
    plsc.subcore_barrier()

    start_rows(0, 0, 0)

    def pair_body(J, carry):
        for m in range(2 * IB):
            c = 2 * IB * J + m
            islot, k = ((m + 1) // IB) % 2, (m + 1) % IB

            if m == IB - 1:
                @pl.when(c + 1 < NCHUNK)
                def _():
                    wait_idx(1)
            if m == 2 * IB - 1:
                @pl.when(c + 1 < NCHUNK)
                def _():
                    wait_idx(0)

            @pl.when(c + 1 < NCHUNK)
            def _():
                start_rows(islot, k, (m + 1) % 2)

            @pl.when(c < NCHUNK)
            def _():
                wait_rows(m % 2)

            @pl.when(jnp.logical_and(2 <= c, c < NCHUNK))
            def _():
                wait_out(m % 2)

            @pl.when(c < NCHUNK)
            def _():
                compute(c, m % 2)

            if m == IB - 1:
                @pl.when(2 * J + 2 < NBLK)
                def _():
                    fetch_idx(2 * J + 2, 0)
            if m == 2 * IB - 1:
                @pl.when(2 * J + 3 < NBLK)
                def _():
                    fetch_idx(2 * J + 3, 1)
        return carry

    lax.fori_loop(0, NBLK // 2, pair_body, jnp.int32(0))
    wait_out(1)
    wait_out(0)


def kernel(h, src, dst):
    pad = NBLK * IB * CHUNK - E_TILE
    src4 = jnp.pad(src.reshape(NW, E_TILE), ((0, 0), (0, pad))
                   ).reshape(NW, NBLK, IB, CHUNK)
    dst4 = jnp.pad(dst.reshape(NW, E_TILE), ((0, 0), (0, pad))
                   ).reshape(NW, NBLK, IB, CHUNK)
    run = pl.kernel(
        _edge_cosine_body,
        mesh=plsc.VectorSubcoreMesh(core_axis_name="c", subcore_axis_name="s"),
        out_type=jax.ShapeDtypeStruct((N_EDGES,), jnp.float32),
        scratch_types=[
            pltpu.VMEM_SHARED((N_NODES, D_FEAT), jnp.float32),
            pltpu.VMEM((2, IB, CHUNK), jnp.int32),
            pltpu.VMEM((2, IB, CHUNK), jnp.int32),
            pltpu.VMEM((2, CHUNK, D_FEAT), jnp.float32),
            pltpu.VMEM((2, CHUNK, D_FEAT), jnp.float32),
            pltpu.VMEM((2, CHUNK), jnp.float32),
            pltpu.SemaphoreType.DMA,
            pltpu.SemaphoreType.DMA,
            pltpu.SemaphoreType.DMA,
            pltpu.SemaphoreType.DMA,
            pltpu.SemaphoreType.DMA,
            pltpu.SemaphoreType.DMA,
        ],
        compiler_params=pltpu.CompilerParams(needs_layout_passes=False),
    )
    return run(h, src4, dst4)
